# baseline (speedup 1.0000x reference)
import jax
import jax.numpy as jnp
from jax.experimental import pallas as pl
from jax.experimental.pallas import tpu as pltpu


def _body(idx_ref, x_ref, o_ref):
    c = x_ref.shape[1]
    k = o_ref.shape[1]
    col = jax.lax.broadcasted_iota(jnp.int32, (c, k), 0)
    onehot = (col == idx_ref[0, :][None, :]).astype(jnp.float32)
    o_ref[...] = jnp.dot(x_ref[...], onehot, preferred_element_type=jnp.float32)


def kernel(x, indices):
    b, s, c = x.shape
    k = indices.shape[0]
    rows = b * s
    x2 = x.reshape(rows, c)
    blk = 1024
    out = pl.pallas_call(
        _body,
        grid=(rows // blk,),
        in_specs=[
            pl.BlockSpec((1, k), lambda i: (0, 0)),
            pl.BlockSpec((blk, c), lambda i: (i, 0)),
        ],
        out_specs=pl.BlockSpec((blk, k), lambda i: (i, 0)),
        out_shape=jax.ShapeDtypeStruct((rows, k), jnp.float32),
    )(indices.reshape(1, k), x2)
    return out.reshape(b, s, k)


# speedup vs baseline: 1.0006x; 1.0006x over previous
import jax
import jax.numpy as jnp
from jax.experimental import pallas as pl
from jax.experimental.pallas import tpu as pltpu


def _body(idx_ref, x_ref, o_ref, onehot_ref):
    c = x_ref.shape[1]
    k = o_ref.shape[1]

    @pl.when(pl.program_id(0) == 0)
    def _():
        col = jax.lax.broadcasted_iota(jnp.int32, (c, k), 0)
        onehot_ref[...] = (col == idx_ref[0, :][None, :]).astype(jnp.float32)

    o_ref[...] = jnp.dot(x_ref[...], onehot_ref[...],
                         preferred_element_type=jnp.float32)


def kernel(x, indices):
    b, s, c = x.shape
    k = indices.shape[0]
    rows = b * s
    x2 = x.reshape(rows, c)
    blk = 512
    out = pl.pallas_call(
        _body,
        grid=(rows // blk,),
        in_specs=[
            pl.BlockSpec((1, k), lambda i: (0, 0)),
            pl.BlockSpec((blk, c), lambda i: (i, 0)),
        ],
        out_specs=pl.BlockSpec((blk, k), lambda i: (i, 0)),
        out_shape=jax.ShapeDtypeStruct((rows, k), jnp.float32),
        scratch_shapes=[pltpu.VMEM((c, k), jnp.float32)],
    )(indices.reshape(1, k), x2)
    return out.reshape(b, s, k)


# speedup vs baseline: 1.0789x; 1.0783x over previous
import functools

import jax
import jax.numpy as jnp
from jax import lax
from jax.experimental import pallas as pl
from jax.experimental.pallas import tpu as pltpu
from jax.experimental.pallas import tpu_sc as plsc


def _sc_body(nrb, k, rbpt, nc, x_hbm, idx_hbm, out_hbm, idx_v, buf_v, sem):
    wid = lax.axis_index("s") * nc + lax.axis_index("c")
    rb0 = wid * rbpt
    pltpu.sync_copy(idx_hbm, idx_v)

    idx_vecs = [idx_v[pl.ds(g * 16, 16)] for g in range(k // 16)]
    t_vecs = [v >> 7 for v in idx_vecs]
    l_vecs = [v & 127 for v in idx_vecs]

    for j in range(k):
        g, m = j // 16, j % 16
        t = t_vecs[g][m]
        l = l_vecs[g][m]
        pltpu.make_async_copy(
            x_hbm.at[pl.ds(rb0, rbpt), t, :, l],
            buf_v.at[:, :, j],
            sem,
        ).start()

    pltpu.make_async_copy(
        x_hbm.at[pl.ds(0, rbpt), 0, :, pl.ds(0, k)], buf_v, sem).wait()

    pltpu.sync_copy(buf_v, out_hbm.at[pl.ds(rb0, rbpt), :, :])


def kernel(x, indices):
    b, s, c = x.shape
    k = indices.shape[0]
    rows = b * s
    nrb = rows // 8
    x4 = x.reshape(nrb, 8, c // 128, 128).transpose(0, 2, 1, 3)

    info = plsc.get_sparse_core_info()
    nc, ns = info.num_cores, info.num_subcores
    nw = nc * ns
    rbpt = nrb // nw

    mesh = plsc.VectorSubcoreMesh(core_axis_name="c", subcore_axis_name="s")
    sc_call = pl.kernel(
        functools.partial(_sc_body, nrb, k, rbpt, nc),
        mesh=mesh,
        out_type=jax.ShapeDtypeStruct((nrb, 8, k), jnp.float32),
        scratch_types=[
            pltpu.VMEM((k,), jnp.int32),
            pltpu.VMEM((rbpt, 8, k), jnp.float32),
            pltpu.SemaphoreType.DMA,
        ],
        compiler_params=pltpu.CompilerParams(use_tc_tiling_on_sc=False),
    )
    out = sc_call(x4, indices)
    return out.reshape(b, s, k)


# speedup vs baseline: 1.1749x; 1.0889x over previous
import functools

import jax
import jax.numpy as jnp
from jax import lax
from jax.experimental import pallas as pl
from jax.experimental.pallas import tpu as pltpu
from jax.experimental.pallas import tpu_sc as plsc

_SC_ROWS = 10752
_TC_BLK = 512


def _sc_body(k, rbpt, nc, x_hbm, idx_hbm, out_hbm, idx_v, buf_v, sem):
    wid = lax.axis_index("s") * nc + lax.axis_index("c")
    rb0 = wid * rbpt
    pltpu.sync_copy(idx_hbm, idx_v)

    idx_vecs = [idx_v[pl.ds(g * 16, 16)] for g in range(k // 16)]
    t_vecs = [v >> 7 for v in idx_vecs]
    l_vecs = [v & 127 for v in idx_vecs]

    for j in range(k):
        g, m = j // 16, j % 16
        t = t_vecs[g][m]
        l = l_vecs[g][m]
        pltpu.make_async_copy(
            x_hbm.at[pl.ds(rb0, rbpt), t, :, l],
            buf_v.at[:, :, j],
            sem,
        ).start()

    pltpu.make_async_copy(
        x_hbm.at[pl.ds(0, rbpt), 0, :, pl.ds(0, k)], buf_v, sem).wait()

    pltpu.sync_copy(buf_v, out_hbm.at[pl.ds(rb0, rbpt), :, :])


def _tc_body(idx_ref, x_ref, o_ref):
    c = x_ref.shape[1]
    k = o_ref.shape[1]
    col = jax.lax.broadcasted_iota(jnp.int32, (c, k), 0)
    onehot = (col == idx_ref[0, :][None, :]).astype(jnp.float32)
    o_ref[...] = jnp.dot(x_ref[...], onehot, preferred_element_type=jnp.float32)


def kernel(x, indices):
    b, s, c = x.shape
    k = indices.shape[0]
    rows = b * s
    nrb = rows // 8
    x2 = x.reshape(rows, c)
    x4 = x.reshape(nrb, 8, c // 128, 128).transpose(0, 2, 1, 3)

    info = plsc.get_sparse_core_info()
    nc, ns = info.num_cores, info.num_subcores
    nw = nc * ns
    nrb_sc = _SC_ROWS // 8
    rbpt = nrb_sc // nw

    mesh = plsc.VectorSubcoreMesh(core_axis_name="c", subcore_axis_name="s")
    sc_call = pl.kernel(
        functools.partial(_sc_body, k, rbpt, nc),
        mesh=mesh,
        out_type=jax.ShapeDtypeStruct((nrb_sc, 8, k), jnp.float32),
        scratch_types=[
            pltpu.VMEM((k,), jnp.int32),
            pltpu.VMEM((rbpt, 8, k), jnp.float32),
            pltpu.SemaphoreType.DMA,
        ],
        compiler_params=pltpu.CompilerParams(use_tc_tiling_on_sc=False),
    )
    out_sc = sc_call(x4, indices)

    tc_rows = rows - _SC_ROWS
    blk0 = _SC_ROWS // _TC_BLK
    out_tc = pl.pallas_call(
        _tc_body,
        grid=(tc_rows // _TC_BLK,),
        in_specs=[
            pl.BlockSpec((1, k), lambda i: (0, 0)),
            pl.BlockSpec((_TC_BLK, c), lambda i: (i + blk0, 0)),
        ],
        out_specs=pl.BlockSpec((_TC_BLK, k), lambda i: (i, 0)),
        out_shape=jax.ShapeDtypeStruct((tc_rows, k), jnp.float32),
    )(indices.reshape(1, k), x2)

    out = jnp.concatenate([out_sc.reshape(_SC_ROWS, k), out_tc], axis=0)
    return out.reshape(b, s, k)
